# baseline (speedup 1.0000x reference)
import functools

import jax
import jax.numpy as jnp
from jax import lax
from jax.experimental import pallas as pl
from jax.experimental.pallas import tpu as pltpu
from jax.experimental.pallas import tpu_sc as plsc

N = 10000
E = 320000
D = 128
A = 16
NSUB = 16
CH = 128
NCHT = E // CH
NCB = NCHT // NSUB
NXTRA = NCHT % NSUB
ZB = 400
NZB = N // ZB
ZR = 40
BR = 1000
GRID = N // BR

_f32 = jnp.float32


def _fill_rows(ref, nrows, ncols16, value):
    v = jnp.full((16,), value, dtype=_f32)

    def body(i, c):
        for j in range(ncols16):
            ref[i, pl.ds(j * 16, 16)] = v
        return c

    lax.fori_loop(0, nrows, body, 0)


def _for_my_blocks(sid, fn):
    for k in range((NZB + NSUB - 1) // NSUB):
        b = sid + k * NSUB
        if (k + 1) * NSUB <= NZB:
            fn(b)
        else:
            @pl.when(b < NZB)
            def _():
                fn(b)


def _zero_start(sid, zbuf, acc, zsem):
    def go(b):
        for k in range(ZB // ZR):
            pltpu.async_copy(zbuf, acc.at[pl.ds(b * ZB + k * ZR, ZR)], zsem)
    _for_my_blocks(sid, go)


def _zero_drain(sid, zbuf, acc, zsem):
    def go(b):
        for k in range(ZB // ZR):
            pltpu.make_async_copy(
                zbuf, acc.at[pl.ds(b * ZB + k * ZR, ZR)], zsem).wait()
    _for_my_blocks(sid, go)


def _flush(sid, acc, outr, zsem):
    _for_my_blocks(sid, lambda b: pltpu.async_copy(
        acc.at[pl.ds(b * ZB, ZB)], outr.at[pl.ds(b * ZB, ZB)], zsem))
    _for_my_blocks(sid, lambda b: pltpu.make_async_copy(
        acc.at[pl.ds(b * ZB, ZB)], outr.at[pl.ds(b * ZB, ZB)], zsem).wait())


def _sc_pre_body(attr101, e101, attr110, e110, attr021, e021, attr030, e030,
                 p101, p110, p021, p030,
                 acc, abuf0, abuf1, idxb0, idxb1, combo0, combo1, zbuf,
                 isem0, isem1, ssem0, ssem1, zsem):
    cid = lax.axis_index("c")
    sid = lax.axis_index("s")
    abuf = (abuf0, abuf1)
    idxb = (idxb0, idxb1)
    combo = (combo0, combo1)
    isem = (isem0, isem1)
    ssem = (ssem0, ssem1)

    _fill_rows(zbuf, ZR, 8, 0.0)
    one = jnp.full((16,), 1.0, dtype=_f32)
    zero = jnp.zeros((16,), dtype=_f32)

    def cb(i, c):
        for k in range(2):
            combo[k][i, pl.ds(16, 16)] = one
            for j in range(2, 8):
                combo[k][i, pl.ds(j * 16, 16)] = zero
        return c

    lax.fori_loop(0, CH, cb, 0)

    def run_type(attr_r, ei_r, opre):
        _zero_start(sid, zbuf, acc, zsem)
        _zero_drain(sid, zbuf, acc, zsem)
        plsc.subcore_barrier()

        def idx_start(c, k):
            b = (c * NSUB + sid) * CH
            pltpu.async_copy(attr_r.at[pl.ds(b * A, CH * A)], abuf[k], isem[k])
            pltpu.async_copy(ei_r.at[:, pl.ds(b, CH)], idxb[k], isem[k])

        def idx_wait(k):
            pltpu.make_async_copy(
                attr_r.at[pl.ds(0, CH * A)], abuf[k], isem[k]).wait()
            pltpu.make_async_copy(
                ei_r.at[:, pl.ds(0, CH)], idxb[k], isem[k]).wait()

        def veccopy(k):
            def cp(i, cc):
                for j in range(8):
                    r = i * 8 + j
                    combo[k][r, pl.ds(0, 16)] = abuf[k][pl.ds(r * 16, 16)]
                return cc
            lax.fori_loop(0, CH // 8, cp, 0)

        def scat_start(k):
            pltpu.async_copy(combo[k], acc.at[idxb[k].at[1]], ssem[k],
                             add=True)

        def scat_wait(k):
            pltpu.make_async_copy(combo[k], acc.at[idxb[k].at[1]],
                                  ssem[k]).wait()

        idx_start(0, 0)
        idx_wait(0)
        veccopy(0)
        scat_start(0)
        idx_start(1, 1)

        def pair(i, carry):
            c1 = 2 * i + 1
            idx_wait(1)
            veccopy(1)
            scat_start(1)
            scat_wait(0)
            idx_start(c1 + 1, 0)
            idx_wait(0)
            veccopy(0)
            scat_start(0)
            scat_wait(1)
            idx_start(c1 + 2, 1)
            return carry

        lax.fori_loop(0, NCB // 2 - 1, pair, 0)
        idx_wait(1)
        veccopy(1)
        scat_start(1)
        scat_wait(0)
        scat_wait(1)

        @pl.when(sid < NXTRA)
        def _():
            idx_start(NCB, 0)
            idx_wait(0)
            veccopy(0)
            scat_start(0)
            scat_wait(0)

        plsc.subcore_barrier()
        _flush(sid, acc, opre, zsem)
        plsc.subcore_barrier()

    @pl.when(cid == 0)
    def _():
        run_type(attr101, e101, p101)
        run_type(attr110, e110, p110)

    @pl.when(cid == 1)
    def _():
        run_type(attr021, e021, p021)
        run_type(attr030, e030, p030)


def _sc_pre(attr101, e101, attr110, e110, attr021, e021, attr030, e030):
    out = tuple(jax.ShapeDtypeStruct((N, D), _f32) for _ in range(4))
    fn = pl.kernel(
        _sc_pre_body,
        out_type=out,
        mesh=plsc.VectorSubcoreMesh(core_axis_name="c", subcore_axis_name="s"),
        scratch_types=[
            pltpu.VMEM_SHARED((N, D), _f32),
            pltpu.VMEM((CH * A,), _f32),
            pltpu.VMEM((CH * A,), _f32),
            pltpu.VMEM((2, CH), jnp.int32),
            pltpu.VMEM((2, CH), jnp.int32),
            pltpu.VMEM((CH, D), _f32),
            pltpu.VMEM((CH, D), _f32),
            pltpu.VMEM((ZR, D), _f32),
            pltpu.SemaphoreType.DMA,
            pltpu.SemaphoreType.DMA,
            pltpu.SemaphoreType.DMA,
            pltpu.SemaphoreType.DMA,
            pltpu.SemaphoreType.DMA,
        ],
    )
    return fn(attr101.reshape(-1), e101, attr110.reshape(-1), e110,
              attr021.reshape(-1), e021, attr030.reshape(-1), e030)


def _sc_seg_body(h0, h1, e101, e110, e021, e030,
                 o101, o110, o021, o030,
                 acc, idxb0, idxb1, rows0, rows1, zbuf,
                 isem0, isem1, gsem0, gsem1, ssem0, ssem1, zsem):
    cid = lax.axis_index("c")
    sid = lax.axis_index("s")
    idxb = (idxb0, idxb1)
    rows = (rows0, rows1)
    isem = (isem0, isem1)
    gsem = (gsem0, gsem1)
    ssem = (ssem0, ssem1)

    _fill_rows(zbuf, ZR, 8, 0.0)

    def run_type(table, ei_r, outr):
        _zero_start(sid, zbuf, acc, zsem)
        _zero_drain(sid, zbuf, acc, zsem)
        plsc.subcore_barrier()

        def idx_start(c, k):
            b = (c * NSUB + sid) * CH
            pltpu.async_copy(ei_r.at[:, pl.ds(b, CH)], idxb[k], isem[k])

        def idx_wait(k):
            pltpu.make_async_copy(
                ei_r.at[:, pl.ds(0, CH)], idxb[k], isem[k]).wait()

        def gather_start(k):
            pltpu.async_copy(table.at[idxb[k].at[0]], rows[k], gsem[k])

        def gather_wait(k):
            pltpu.make_async_copy(table.at[idxb[k].at[0]], rows[k],
                                  gsem[k]).wait()

        def scat_start(k):
            pltpu.async_copy(rows[k], acc.at[idxb[k].at[1]], ssem[k],
                             add=True)

        def scat_wait(k):
            pltpu.make_async_copy(rows[k], acc.at[idxb[k].at[1]],
                                  ssem[k]).wait()

        idx_start(0, 0)
        idx_wait(0)
        gather_start(0)
        idx_start(1, 1)

        def pair(i, carry):
            c0 = 2 * i
            gather_wait(0)
            scat_start(0)
            idx_wait(1)
            gather_start(1)
            scat_wait(0)
            idx_start(c0 + 2, 0)
            gather_wait(1)
            scat_start(1)
            idx_wait(0)
            gather_start(0)
            scat_wait(1)
            idx_start(c0 + 3, 1)
            return carry

        lax.fori_loop(0, NCB // 2 - 1, pair, 0)
        gather_wait(0)
        scat_start(0)
        idx_wait(1)
        gather_start(1)
        scat_wait(0)
        gather_wait(1)
        scat_start(1)
        scat_wait(1)

        @pl.when(sid < NXTRA)
        def _():
            idx_start(NCB, 0)
            idx_wait(0)
            gather_start(0)
            gather_wait(0)
            scat_start(0)
            scat_wait(0)

        plsc.subcore_barrier()
        _flush(sid, acc, outr, zsem)
        plsc.subcore_barrier()

    @pl.when(cid == 0)
    def _():
        run_type(h1, e101, o101)
        run_type(h1, e110, o110)

    @pl.when(cid == 1)
    def _():
        run_type(h0, e021, o021)
        run_type(h0, e030, o030)


def _sc_seg(h0, h1, e101, e110, e021, e030):
    out = tuple(jax.ShapeDtypeStruct((N, D), _f32) for _ in range(4))
    fn = pl.kernel(
        _sc_seg_body,
        out_type=out,
        mesh=plsc.VectorSubcoreMesh(core_axis_name="c", subcore_axis_name="s"),
        scratch_types=[
            pltpu.VMEM_SHARED((N, D), _f32),
            pltpu.VMEM((2, CH), jnp.int32),
            pltpu.VMEM((2, CH), jnp.int32),
            pltpu.VMEM((CH, D), _f32),
            pltpu.VMEM((CH, D), _f32),
            pltpu.VMEM((ZR, D), _f32),
            pltpu.SemaphoreType.DMA,
            pltpu.SemaphoreType.DMA,
            pltpu.SemaphoreType.DMA,
            pltpu.SemaphoreType.DMA,
            pltpu.SemaphoreType.DMA,
            pltpu.SemaphoreType.DMA,
            pltpu.SemaphoreType.DMA,
        ],
    )
    return fn(h0, h1, e101, e110, e021, e030)


def _row_spec(w):
    return pl.BlockSpec((BR, w), lambda i: (i, 0))


def _full_spec(shape):
    return pl.BlockSpec(shape, lambda i: tuple(0 for _ in shape))


def _tc_h_body(x0r, x1r, wr, br, h0r, h1r):
    w = wr[...]
    b = br[...]
    h0r[...] = jnp.dot(x0r[...], w, preferred_element_type=_f32) + b
    h1r[...] = jnp.dot(x1r[...], w, preferred_element_type=_f32) + b


def _tc_h(x0, x1, W_x, b_x):
    return pl.pallas_call(
        _tc_h_body,
        grid=(GRID,),
        in_specs=[_row_spec(128), _row_spec(128),
                  _full_spec((128, D)), _full_spec((1, D))],
        out_specs=[_row_spec(D), _row_spec(D)],
        out_shape=[jax.ShapeDtypeStruct((N, D), _f32)] * 2,
    )(x0, x1, W_x, b_x.reshape(1, D))


def _tc_dense_body(s101r, s021r, s110r, s030r,
                   p101r, p021r, p110r, p030r,
                   h1r, wer, ber, wg1r, bg1r, wg2r, bg2r,
                   wl110r, bl110r, wl021r, bl021r, wl030r, bl030r,
                   y0r, y1r, str_):
    we = wer[...]
    be = ber[...]

    def aggr(sr, pr):
        p = pr[...]
        return (sr[...] + jnp.dot(p[:, :A], we, preferred_element_type=_f32)
                + p[:, A:A + 1] * be)

    t = aggr(s101r, p101r) + 1.1 * h1r[...]
    u = jnp.maximum(
        jnp.dot(t, wg1r[...], preferred_element_type=_f32) + bg1r[...], 0.0)
    out_gin = jnp.dot(u, wg2r[...], preferred_element_type=_f32) + bg2r[...]
    out_021 = (jnp.dot(aggr(s021r, p021r), wl021r[...],
                       preferred_element_type=_f32) + bl021r[...]) * 0.1
    y1 = (out_gin + out_021) * 0.5

    out_110 = (jnp.dot(aggr(s110r, p110r), wl110r[...],
                       preferred_element_type=_f32) + bl110r[...]) * 0.1
    out_030 = (jnp.dot(aggr(s030r, p030r), wl030r[...],
                       preferred_element_type=_f32) + bl030r[...]) * 0.1
    y0 = (out_110 + out_030) * 0.5

    y0r[...] = y0
    y1r[...] = y1
    st = jnp.stack([jnp.sum(y0, 0), jnp.sum(y0 * y0, 0),
                    jnp.sum(y1, 0), jnp.sum(y1 * y1, 0)])
    str_[...] = st.reshape(1, 4, D)


def _tc_dense(s101, s021, s110, s030, p101, p021, p110, p030,
              h1, W_e, b_e,
              W_gin1, b_gin1, W_gin2, b_gin2,
              W_l110, b_l110, W_l021, b_l021, W_l030, b_l030):
    return pl.pallas_call(
        _tc_dense_body,
        grid=(GRID,),
        in_specs=[_row_spec(D)] * 4 + [_row_spec(D)] * 4 + [_row_spec(D)]
        + [_full_spec((A, D)), _full_spec((1, D)),
           _full_spec((D, 2 * D)), _full_spec((1, 2 * D)),
           _full_spec((2 * D, D)), _full_spec((1, D)),
           _full_spec((D, D)), _full_spec((1, D)),
           _full_spec((D, D)), _full_spec((1, D)),
           _full_spec((D, D)), _full_spec((1, D))],
        out_specs=[_row_spec(D), _row_spec(D),
                   pl.BlockSpec((1, 4, D), lambda i: (i, 0, 0))],
        out_shape=[jax.ShapeDtypeStruct((N, D), _f32),
                   jax.ShapeDtypeStruct((N, D), _f32),
                   jax.ShapeDtypeStruct((GRID, 4, D), _f32)],
    )(s101, s021, s110, s030, p101, p021, p110, p030, h1,
      W_e, b_e.reshape(1, D), W_gin1, b_gin1.reshape(1, 2 * D),
      W_gin2, b_gin2.reshape(1, D), W_l110, b_l110.reshape(1, D),
      W_l021, b_l021.reshape(1, D), W_l030, b_l030.reshape(1, D))


def _tc_bn_body(do_relu, y0r, y1r, str_, gr, br, h0r, h1r):
    st = jnp.sum(str_[...], axis=0)
    n = jnp.float32(N)
    mu0 = st[0:1, :] / n
    var0 = st[1:2, :] / n - mu0 * mu0
    mu1 = st[2:3, :] / n
    var1 = st[3:4, :] / n - mu1 * mu1
    g = gr[...]
    b = br[...]
    h0 = g * (y0r[...] - mu0) / jnp.sqrt(var0 + 1e-5) + b
    h1 = g * (y1r[...] - mu1) / jnp.sqrt(var1 + 1e-5) + b
    if do_relu:
        h0 = jnp.maximum(h0, 0.0)
        h1 = jnp.maximum(h1, 0.0)
    h0r[...] = h0
    h1r[...] = h1


def _tc_bn(y0, y1, st, gamma, beta, do_relu):
    return pl.pallas_call(
        functools.partial(_tc_bn_body, do_relu),
        grid=(GRID,),
        in_specs=[_row_spec(D), _row_spec(D), _full_spec((GRID, 4, D)),
                  _full_spec((1, D)), _full_spec((1, D))],
        out_specs=[_row_spec(D), _row_spec(D)],
        out_shape=[jax.ShapeDtypeStruct((N, D), _f32)] * 2,
    )(y0, y1, st, gamma.reshape(1, D), beta.reshape(1, D))


def kernel(x0, x1, edge_index_101, edge_index_110, edge_index_021,
           edge_index_030, edge_attr_101, edge_attr_110, edge_attr_021,
           edge_attr_030, W_x, b_x, W_e, b_e, W_gin1, b_gin1, W_gin2, b_gin2,
           W_l110, b_l110, W_l021, b_l021, W_l030, b_l030, bn_gamma, bn_beta):
    p101, p110, p021, p030 = _sc_pre(
        edge_attr_101, edge_index_101, edge_attr_110, edge_index_110,
        edge_attr_021, edge_index_021, edge_attr_030, edge_index_030)

    h0, h1 = _tc_h(x0, x1, W_x, b_x)

    for layer in range(2):
        g101, g110, g021, g030 = _sc_seg(
            h0, h1, edge_index_101, edge_index_110,
            edge_index_021, edge_index_030)
        y0, y1, st = _tc_dense(
            g101, g021, g110, g030, p101, p021, p110, p030,
            h1, W_e, b_e,
            W_gin1, b_gin1, W_gin2, b_gin2,
            W_l110, b_l110, W_l021, b_l021, W_l030, b_l030)
        h0, h1 = _tc_bn(y0, y1, st, bn_gamma[layer], bn_beta[layer],
                        do_relu=(layer == 0))

    return jnp.concatenate([h0, h1], axis=0)


# speedup vs baseline: 1.0310x; 1.0310x over previous
import functools

import jax
import jax.numpy as jnp
from jax import lax
from jax.experimental import pallas as pl
from jax.experimental.pallas import tpu as pltpu
from jax.experimental.pallas import tpu_sc as plsc

N = 10000
E = 320000
D = 128
A = 16
NSUB = 16
CH = 128
NCHT = E // CH
NCB = NCHT // NSUB
NXTRA = NCHT % NSUB
ZB = 400
NZB = N // ZB
ZR = 40
BR = 1000
GRID = N // BR

_f32 = jnp.float32


def _fill_rows(ref, nrows, ncols16, value):
    v = jnp.full((16,), value, dtype=_f32)

    def body(i, c):
        for j in range(ncols16):
            ref[i, pl.ds(j * 16, 16)] = v
        return c

    lax.fori_loop(0, nrows, body, 0)


def _for_my_blocks(sid, fn):
    for k in range((NZB + NSUB - 1) // NSUB):
        b = sid + k * NSUB
        if (k + 1) * NSUB <= NZB:
            fn(b)
        else:
            @pl.when(b < NZB)
            def _():
                fn(b)


def _zero_start(sid, zbuf, acc, zsem):
    def go(b):
        for k in range(ZB // ZR):
            pltpu.async_copy(zbuf, acc.at[pl.ds(b * ZB + k * ZR, ZR)], zsem)
    _for_my_blocks(sid, go)


def _zero_drain(sid, zbuf, acc, zsem):
    def go(b):
        for k in range(ZB // ZR):
            pltpu.make_async_copy(
                zbuf, acc.at[pl.ds(b * ZB + k * ZR, ZR)], zsem).wait()
    _for_my_blocks(sid, go)


def _flush(sid, acc, outr, zsem):
    _for_my_blocks(sid, lambda b: pltpu.async_copy(
        acc.at[pl.ds(b * ZB, ZB)], outr.at[pl.ds(b * ZB, ZB)], zsem))
    _for_my_blocks(sid, lambda b: pltpu.make_async_copy(
        acc.at[pl.ds(b * ZB, ZB)], outr.at[pl.ds(b * ZB, ZB)], zsem).wait())


def _sc_pre_body(attr101, e101, attr110, e110, attr021, e021, attr030, e030,
                 p101, p110, p021, p030,
                 acc, abuf0, abuf1, idxb0, idxb1, combo0, combo1, zbuf,
                 isem0, isem1, ssem0, ssem1, zsem):
    cid = lax.axis_index("c")
    sid = lax.axis_index("s")
    abuf = (abuf0, abuf1)
    idxb = (idxb0, idxb1)
    combo = (combo0, combo1)
    isem = (isem0, isem1)
    ssem = (ssem0, ssem1)

    _fill_rows(zbuf, ZR, 8, 0.0)
    one = jnp.full((16,), 1.0, dtype=_f32)
    zero = jnp.zeros((16,), dtype=_f32)

    def cb(i, c):
        for k in range(2):
            combo[k][i, pl.ds(16, 16)] = one
            for j in range(2, 8):
                combo[k][i, pl.ds(j * 16, 16)] = zero
        return c

    lax.fori_loop(0, CH, cb, 0)

    def run_type(attr_r, ei_r, opre):
        _zero_start(sid, zbuf, acc, zsem)
        _zero_drain(sid, zbuf, acc, zsem)
        plsc.subcore_barrier()

        def idx_start(c, k):
            b = (c * NSUB + sid) * CH
            pltpu.async_copy(attr_r.at[pl.ds(b * A, CH * A)], abuf[k], isem[k])
            pltpu.async_copy(ei_r.at[:, pl.ds(b, CH)], idxb[k], isem[k])

        def idx_wait(k):
            pltpu.make_async_copy(
                attr_r.at[pl.ds(0, CH * A)], abuf[k], isem[k]).wait()
            pltpu.make_async_copy(
                ei_r.at[:, pl.ds(0, CH)], idxb[k], isem[k]).wait()

        def veccopy(k):
            def cp(i, cc):
                for j in range(8):
                    r = i * 8 + j
                    combo[k][r, pl.ds(0, 16)] = abuf[k][pl.ds(r * 16, 16)]
                return cc
            lax.fori_loop(0, CH // 8, cp, 0)

        def scat_start(k):
            pltpu.async_copy(combo[k], acc.at[idxb[k].at[1]], ssem[k],
                             add=True)

        def scat_wait(k):
            pltpu.make_async_copy(combo[k], acc.at[idxb[k].at[1]],
                                  ssem[k]).wait()

        idx_start(0, 0)
        idx_wait(0)
        veccopy(0)
        scat_start(0)
        idx_start(1, 1)

        def pair(i, carry):
            c1 = 2 * i + 1
            idx_wait(1)
            veccopy(1)
            scat_start(1)
            scat_wait(0)
            idx_start(c1 + 1, 0)
            idx_wait(0)
            veccopy(0)
            scat_start(0)
            scat_wait(1)
            idx_start(c1 + 2, 1)
            return carry

        lax.fori_loop(0, NCB // 2 - 1, pair, 0)
        idx_wait(1)
        veccopy(1)
        scat_start(1)
        scat_wait(0)
        scat_wait(1)

        @pl.when(sid < NXTRA)
        def _():
            idx_start(NCB, 0)
            idx_wait(0)
            veccopy(0)
            scat_start(0)
            scat_wait(0)

        plsc.subcore_barrier()
        _flush(sid, acc, opre, zsem)
        plsc.subcore_barrier()

    @pl.when(cid == 0)
    def _():
        run_type(attr101, e101, p101)
        run_type(attr110, e110, p110)

    @pl.when(cid == 1)
    def _():
        run_type(attr021, e021, p021)
        run_type(attr030, e030, p030)


def _sc_pre(attr101, e101, attr110, e110, attr021, e021, attr030, e030):
    out = tuple(jax.ShapeDtypeStruct((N, D), _f32) for _ in range(4))
    fn = pl.kernel(
        _sc_pre_body,
        out_type=out,
        mesh=plsc.VectorSubcoreMesh(core_axis_name="c", subcore_axis_name="s"),
        scratch_types=[
            pltpu.VMEM_SHARED((N, D), _f32),
            pltpu.VMEM((CH * A,), _f32),
            pltpu.VMEM((CH * A,), _f32),
            pltpu.VMEM((2, CH), jnp.int32),
            pltpu.VMEM((2, CH), jnp.int32),
            pltpu.VMEM((CH, D), _f32),
            pltpu.VMEM((CH, D), _f32),
            pltpu.VMEM((ZR, D), _f32),
            pltpu.SemaphoreType.DMA,
            pltpu.SemaphoreType.DMA,
            pltpu.SemaphoreType.DMA,
            pltpu.SemaphoreType.DMA,
            pltpu.SemaphoreType.DMA,
        ],
    )
    return fn(attr101.reshape(-1), e101, attr110.reshape(-1), e110,
              attr021.reshape(-1), e021, attr030.reshape(-1), e030)


def _sc_seg_body(h0, h1, e101, e110, e021, e030,
                 o101, o110, o021, o030,
                 acc, idxb0, idxb1, rows0, rows1, zbuf,
                 isem0, isem1, gsem0, gsem1, ssem0, ssem1, zsem):
    cid = lax.axis_index("c")
    sid = lax.axis_index("s")
    idxb = (idxb0, idxb1)
    rows = (rows0, rows1)
    isem = (isem0, isem1)
    gsem = (gsem0, gsem1)
    ssem = (ssem0, ssem1)

    _fill_rows(zbuf, ZR, 8, 0.0)

    def run_type(table, ei_r, outr):
        _zero_start(sid, zbuf, acc, zsem)
        _zero_drain(sid, zbuf, acc, zsem)
        plsc.subcore_barrier()

        def idx_start(c, k):
            b = (c * NSUB + sid) * CH
            pltpu.async_copy(ei_r.at[:, pl.ds(b, CH)], idxb[k], isem[k])

        def idx_wait(k):
            pltpu.make_async_copy(
                ei_r.at[:, pl.ds(0, CH)], idxb[k], isem[k]).wait()

        def gather_start(k):
            pltpu.async_copy(table.at[idxb[k].at[0]], rows[k], gsem[k])

        def gather_wait(k):
            pltpu.make_async_copy(table.at[idxb[k].at[0]], rows[k],
                                  gsem[k]).wait()

        def scat_start(k):
            pltpu.async_copy(rows[k], acc.at[idxb[k].at[1]], ssem[k],
                             add=True)

        def scat_wait(k):
            pltpu.make_async_copy(rows[k], acc.at[idxb[k].at[1]],
                                  ssem[k]).wait()

        idx_start(0, 0)
        idx_wait(0)
        gather_start(0)
        idx_start(1, 1)

        def pair(i, carry):
            c0 = 2 * i
            idx_wait(1)
            gather_start(1)
            gather_wait(0)
            scat_start(0)
            scat_wait(0)
            idx_start(c0 + 2, 0)
            idx_wait(0)
            gather_start(0)
            gather_wait(1)
            scat_start(1)
            scat_wait(1)
            idx_start(c0 + 3, 1)
            return carry

        lax.fori_loop(0, NCB // 2 - 1, pair, 0)
        idx_wait(1)
        gather_start(1)
        gather_wait(0)
        scat_start(0)
        scat_wait(0)
        gather_wait(1)
        scat_start(1)
        scat_wait(1)

        @pl.when(sid < NXTRA)
        def _():
            idx_start(NCB, 0)
            idx_wait(0)
            gather_start(0)
            gather_wait(0)
            scat_start(0)
            scat_wait(0)

        plsc.subcore_barrier()
        _flush(sid, acc, outr, zsem)
        plsc.subcore_barrier()

    @pl.when(cid == 0)
    def _():
        run_type(h1, e101, o101)
        run_type(h1, e110, o110)

    @pl.when(cid == 1)
    def _():
        run_type(h0, e021, o021)
        run_type(h0, e030, o030)


def _sc_seg(h0, h1, e101, e110, e021, e030):
    out = tuple(jax.ShapeDtypeStruct((N, D), _f32) for _ in range(4))
    fn = pl.kernel(
        _sc_seg_body,
        out_type=out,
        mesh=plsc.VectorSubcoreMesh(core_axis_name="c", subcore_axis_name="s"),
        scratch_types=[
            pltpu.VMEM_SHARED((N, D), _f32),
            pltpu.VMEM((2, CH), jnp.int32),
            pltpu.VMEM((2, CH), jnp.int32),
            pltpu.VMEM((CH, D), _f32),
            pltpu.VMEM((CH, D), _f32),
            pltpu.VMEM((ZR, D), _f32),
            pltpu.SemaphoreType.DMA,
            pltpu.SemaphoreType.DMA,
            pltpu.SemaphoreType.DMA,
            pltpu.SemaphoreType.DMA,
            pltpu.SemaphoreType.DMA,
            pltpu.SemaphoreType.DMA,
            pltpu.SemaphoreType.DMA,
        ],
    )
    return fn(h0, h1, e101, e110, e021, e030)


def _row_spec(w):
    return pl.BlockSpec((BR, w), lambda i: (i, 0))


def _full_spec(shape):
    return pl.BlockSpec(shape, lambda i: tuple(0 for _ in shape))


def _tc_h_body(x0r, x1r, wr, br, h0r, h1r):
    w = wr[...]
    b = br[...]
    h0r[...] = jnp.dot(x0r[...], w, preferred_element_type=_f32) + b
    h1r[...] = jnp.dot(x1r[...], w, preferred_element_type=_f32) + b


def _tc_h(x0, x1, W_x, b_x):
    return pl.pallas_call(
        _tc_h_body,
        grid=(GRID,),
        in_specs=[_row_spec(128), _row_spec(128),
                  _full_spec((128, D)), _full_spec((1, D))],
        out_specs=[_row_spec(D), _row_spec(D)],
        out_shape=[jax.ShapeDtypeStruct((N, D), _f32)] * 2,
    )(x0, x1, W_x, b_x.reshape(1, D))


def _tc_dense_body(s101r, s021r, s110r, s030r,
                   p101r, p021r, p110r, p030r,
                   h1r, wer, ber, wg1r, bg1r, wg2r, bg2r,
                   wl110r, bl110r, wl021r, bl021r, wl030r, bl030r,
                   y0r, y1r, str_):
    we = wer[...]
    be = ber[...]

    def aggr(sr, pr):
        p = pr[...]
        return (sr[...] + jnp.dot(p[:, :A], we, preferred_element_type=_f32)
                + p[:, A:A + 1] * be)

    t = aggr(s101r, p101r) + 1.1 * h1r[...]
    u = jnp.maximum(
        jnp.dot(t, wg1r[...], preferred_element_type=_f32) + bg1r[...], 0.0)
    out_gin = jnp.dot(u, wg2r[...], preferred_element_type=_f32) + bg2r[...]
    out_021 = (jnp.dot(aggr(s021r, p021r), wl021r[...],
                       preferred_element_type=_f32) + bl021r[...]) * 0.1
    y1 = (out_gin + out_021) * 0.5

    out_110 = (jnp.dot(aggr(s110r, p110r), wl110r[...],
                       preferred_element_type=_f32) + bl110r[...]) * 0.1
    out_030 = (jnp.dot(aggr(s030r, p030r), wl030r[...],
                       preferred_element_type=_f32) + bl030r[...]) * 0.1
    y0 = (out_110 + out_030) * 0.5

    y0r[...] = y0
    y1r[...] = y1
    st = jnp.stack([jnp.sum(y0, 0), jnp.sum(y0 * y0, 0),
                    jnp.sum(y1, 0), jnp.sum(y1 * y1, 0)])
    str_[...] = st.reshape(1, 4, D)


def _tc_dense(s101, s021, s110, s030, p101, p021, p110, p030,
              h1, W_e, b_e,
              W_gin1, b_gin1, W_gin2, b_gin2,
              W_l110, b_l110, W_l021, b_l021, W_l030, b_l030):
    return pl.pallas_call(
        _tc_dense_body,
        grid=(GRID,),
        in_specs=[_row_spec(D)] * 4 + [_row_spec(D)] * 4 + [_row_spec(D)]
        + [_full_spec((A, D)), _full_spec((1, D)),
           _full_spec((D, 2 * D)), _full_spec((1, 2 * D)),
           _full_spec((2 * D, D)), _full_spec((1, D)),
           _full_spec((D, D)), _full_spec((1, D)),
           _full_spec((D, D)), _full_spec((1, D)),
           _full_spec((D, D)), _full_spec((1, D))],
        out_specs=[_row_spec(D), _row_spec(D),
                   pl.BlockSpec((1, 4, D), lambda i: (i, 0, 0))],
        out_shape=[jax.ShapeDtypeStruct((N, D), _f32),
                   jax.ShapeDtypeStruct((N, D), _f32),
                   jax.ShapeDtypeStruct((GRID, 4, D), _f32)],
    )(s101, s021, s110, s030, p101, p021, p110, p030, h1,
      W_e, b_e.reshape(1, D), W_gin1, b_gin1.reshape(1, 2 * D),
      W_gin2, b_gin2.reshape(1, D), W_l110, b_l110.reshape(1, D),
      W_l021, b_l021.reshape(1, D), W_l030, b_l030.reshape(1, D))


def _tc_bn_body(do_relu, y0r, y1r, str_, gr, br, h0r, h1r):
    st = jnp.sum(str_[...], axis=0)
    n = jnp.float32(N)
    mu0 = st[0:1, :] / n
    var0 = st[1:2, :] / n - mu0 * mu0
    mu1 = st[2:3, :] / n
    var1 = st[3:4, :] / n - mu1 * mu1
    g = gr[...]
    b = br[...]
    h0 = g * (y0r[...] - mu0) / jnp.sqrt(var0 + 1e-5) + b
    h1 = g * (y1r[...] - mu1) / jnp.sqrt(var1 + 1e-5) + b
    if do_relu:
        h0 = jnp.maximum(h0, 0.0)
        h1 = jnp.maximum(h1, 0.0)
    h0r[...] = h0
    h1r[...] = h1


def _tc_bn(y0, y1, st, gamma, beta, do_relu):
    return pl.pallas_call(
        functools.partial(_tc_bn_body, do_relu),
        grid=(GRID,),
        in_specs=[_row_spec(D), _row_spec(D), _full_spec((GRID, 4, D)),
                  _full_spec((1, D)), _full_spec((1, D))],
        out_specs=[_row_spec(D), _row_spec(D)],
        out_shape=[jax.ShapeDtypeStruct((N, D), _f32)] * 2,
    )(y0, y1, st, gamma.reshape(1, D), beta.reshape(1, D))


def kernel(x0, x1, edge_index_101, edge_index_110, edge_index_021,
           edge_index_030, edge_attr_101, edge_attr_110, edge_attr_021,
           edge_attr_030, W_x, b_x, W_e, b_e, W_gin1, b_gin1, W_gin2, b_gin2,
           W_l110, b_l110, W_l021, b_l021, W_l030, b_l030, bn_gamma, bn_beta):
    p101, p110, p021, p030 = _sc_pre(
        edge_attr_101, edge_index_101, edge_attr_110, edge_index_110,
        edge_attr_021, edge_index_021, edge_attr_030, edge_index_030)

    h0, h1 = _tc_h(x0, x1, W_x, b_x)

    for layer in range(2):
        g101, g110, g021, g030 = _sc_seg(
            h0, h1, edge_index_101, edge_index_110,
            edge_index_021, edge_index_030)
        y0, y1, st = _tc_dense(
            g101, g021, g110, g030, p101, p021, p110, p030,
            h1, W_e, b_e,
            W_gin1, b_gin1, W_gin2, b_gin2,
            W_l110, b_l110, W_l021, b_l021, W_l030, b_l030)
        h0, h1 = _tc_bn(y0, y1, st, bn_gamma[layer], bn_beta[layer],
                        do_relu=(layer == 0))

    return jnp.concatenate([h0, h1], axis=0)
